# baseline (speedup 1.0000x reference)
import functools

import jax
import jax.numpy as jnp
from jax import lax
from jax.experimental import pallas as pl
from jax.experimental.pallas import tpu as pltpu
from jax.experimental.pallas import tpu_sc as plsc

_S = 64.0
_MS = 0.35 * 64.0

_ROWS = 1024
_COLS = 100000
_CW = 2048
_NK = 48
_TAIL0 = _NK * _CW
_TAILW = _COLS - _TAIL0
_L = 16
_NBUF = 3


def _make_sc_kernel():
    info = plsc.get_sparse_core_info()
    nw = info.num_cores * info.num_subcores
    rpw = _ROWS // nw
    ngrp = rpw // 8
    total = ngrp * _NK
    mesh = plsc.VectorSubcoreMesh(core_axis_name="c", subcore_axis_name="s")

    @functools.partial(
        pl.kernel,
        out_type=jax.ShapeDtypeStruct((_ROWS, _COLS), jnp.float32),
        mesh=mesh,
        compiler_params=pltpu.CompilerParams(needs_layout_passes=False),
        scratch_types=(
            [pltpu.VMEM((rpw,), jnp.int32)]
            + [pltpu.VMEM((8, _CW), jnp.float32) for _ in range(2 * _NBUF)]
            + [pltpu.VMEM((8, _TAILW), jnp.float32)]
            + [pltpu.SemaphoreType.DMA for _ in range(2 * _NBUF)]
        ),
    )
    def sc_kernel(x_hbm, lab_hbm, out_hbm,
                  lab_v, ib0, ib1, ib2, ob0, ob1, ob2, tbuf,
                  is0, is1, is2, os0, os1, os2):
        ibufs, obufs = [ib0, ib1, ib2], [ob0, ob1, ob2]
        isems, osems = [is0, is1, is2], [os0, os1, os2]
        cid = lax.axis_index("c")
        sid = lax.axis_index("s")
        wid = sid * info.num_cores + cid
        row0 = wid * rpw
        pltpu.sync_copy(lab_hbm.at[pl.ds(row0, rpw)], lab_v)

        def hbm_chunk(ref, t):
            rg = t // _NK
            k = t % _NK
            return ref.at[pl.ds(row0 + rg * 8, 8), pl.ds(k * _CW, _CW)]

        def in_copy(t, b):
            return pltpu.make_async_copy(hbm_chunk(x_hbm, t), ibufs[b],
                                         isems[b])

        def out_copy(t, b):
            return pltpu.make_async_copy(obufs[b], hbm_chunk(out_hbm, t),
                                         osems[b])

        def fixup(buf, rg, c0, cw):
            g16 = rg // 2
            labs = lab_v[pl.ds(g16 * _L, _L)]
            lrow = lax.iota(jnp.int32, _L) + g16 * _L
            base = rg * 8
            lanes = lax.iota(jnp.int32, _L)
            for rr in range(8):
                m_rr = (lrow == base + rr) & (labs >= c0) & (labs < c0 + cw)
                sel = jnp.max(jnp.where(m_rr, labs, -1))

                @pl.when(sel >= 0)
                def _():
                    c = sel - c0
                    start = (c // _L) * _L
                    lane = c % _L
                    v = buf[rr, pl.ds(start, _L)]
                    buf[rr, pl.ds(start, _L)] = \
                        jnp.where(lanes == lane, v - _MS, v)

        def compute(t, b):
            ib, ob = ibufs[b], obufs[b]

            def body(i, c):
                for rr in range(8):
                    ob[rr, pl.ds(i * _L, _L)] = ib[rr, pl.ds(i * _L, _L)] * _S
                return c

            lax.fori_loop(0, _CW // _L, body, 0)
            fixup(ob, t // _NK, (t % _NK) * _CW, _CW)

        for b in range(_NBUF):
            in_copy(b, b).start()

        def outer(tt, carry):
            for b in range(_NBUF):
                t = tt * _NBUF + b

                @pl.when(t >= _NBUF)
                def _():
                    out_copy(t - _NBUF, b).wait()

                in_copy(t, b).wait()
                compute(t, b)
                out_copy(t, b).start()

                @pl.when(t + _NBUF < total)
                def _():
                    in_copy(t + _NBUF, b).start()

            return carry

        lax.fori_loop(0, total // _NBUF, outer, 0)

        for b in range(_NBUF):
            out_copy(total - _NBUF + b, (total - _NBUF + b) % _NBUF).wait()

        def tail(rg, carry):
            r = row0 + rg * 8
            pltpu.sync_copy(
                x_hbm.at[pl.ds(r, 8), pl.ds(_TAIL0, _TAILW)], tbuf)

            def body(i, c):
                for rr in range(8):
                    tbuf[rr, pl.ds(i * _L, _L)] = \
                        tbuf[rr, pl.ds(i * _L, _L)] * _S
                return c

            lax.fori_loop(0, _TAILW // _L, body, 0)
            fixup(tbuf, rg, _TAIL0, _TAILW)
            pltpu.sync_copy(
                tbuf, out_hbm.at[pl.ds(r, 8), pl.ds(_TAIL0, _TAILW)])
            return carry

        lax.fori_loop(0, ngrp, tail, 0)

    return sc_kernel


_sc_kernel = _make_sc_kernel()


@jax.jit
def kernel(logits, labels):
    return _sc_kernel(logits, labels.astype(jnp.int32))


# speedup vs baseline: 4.1702x; 4.1702x over previous
import jax
import jax.numpy as jnp
from jax.experimental import pallas as pl

_S = 64.0
_MS = 0.35 * 64.0

_BLK = 2048


def _scale_body(lab_ref, x_ref, o_ref):
    j = pl.program_id(0)
    x = x_ref[...]
    lab = lab_ref[0, :]
    row = jax.lax.broadcasted_iota(jnp.int32, x.shape, 0) + j * _BLK
    delta = jnp.where(row == lab[None, :], -_MS, 0.0).astype(x.dtype)
    o_ref[...] = x * _S + delta


@jax.jit
def kernel(logits, labels):
    rows, cols = logits.shape
    xt = logits.T
    lab2d = jnp.broadcast_to(labels.astype(jnp.int32)[None, :], (8, rows))
    grid = pl.cdiv(cols, _BLK)
    out_t = pl.pallas_call(
        _scale_body,
        grid=(grid,),
        in_specs=[
            pl.BlockSpec((8, rows), lambda j: (0, 0)),
            pl.BlockSpec((_BLK, rows), lambda j: (j, 0)),
        ],
        out_specs=pl.BlockSpec((_BLK, rows), lambda j: (j, 0)),
        out_shape=jax.ShapeDtypeStruct((cols, rows), logits.dtype),
    )(lab2d, xt)
    return out_t.T
